# baseline (speedup 1.0000x reference)
import jax
import jax.numpy as jnp
from jax import lax
from jax.experimental import pallas as pl
from jax.experimental.pallas import tpu as pltpu
from jax.experimental.pallas import tpu_sc as plsc

N_P, N_G, D, E, L = 50000, 10000, 64, 800000, 100000

NC, NS = 2, 16
P_PAD = 50176
Q = P_PAD // 4
E_CH = 512
E_PAD = 802816
CH = 448
G_CPT = E_PAD // (NC * NS * CH)
QROWS = 24
EPTQ = QROWS * CH
QCAP = EPTQ - 2 * CH
L_TILE = 3136
L_CH = 448
L_PAD = 32 * L_TILE

_MESH = plsc.VectorSubcoreMesh(
    core_axis_name="c", subcore_axis_name="s", num_cores=NC, num_subcores=NS)
_SC_PARAMS = pltpu.CompilerParams(
    use_tc_tiling_on_sc=False, needs_layout_passes=False)


def _iota16():
  return lax.iota(jnp.int32, 16)


def _acc_rows(own):
  return -(-(own + 16) // 128) * 128


def _out_rows(own):
  r = own // 16
  return own if r % 8 == 0 else _acc_rows(own)


def _part_body(srcp, dstp, psq, pdq, cnts,
               sidx, didx, bs0, bd0, bs1, bd1, bs2, bd2, bs3, bd3, cv):
  c = lax.axis_index("c")
  s = lax.axis_index("s")
  t = c * NS + s
  ept = E_PAD // 32
  ebase = t * ept
  n_chunks = ept // E_CH
  bufs = ((bs0, bd0), (bs1, bd1), (bs2, bd2), (bs3, bd3))

  def chunk(i, pos):
    eb = pl.multiple_of(ebase + i * E_CH, 32)
    pltpu.sync_copy((srcp.at[pl.ds(eb, E_CH)], dstp.at[pl.ds(eb, E_CH)]),
                    (sidx, didx))
    out_pos = []
    for q in range(4):
      pq = pos[q]
      bqs, bqd = bufs[q]
      for j in range(E_CH // 16):
        sv = sidx[pl.ds(j * 16, 16)]
        dv = didx[pl.ds(j * 16, 16)]
        loc = dv - q * Q
        ok = (loc >= 0) & (loc < Q)
        cs = plsc.cumsum(jnp.where(ok, 1, 0))
        idx = pq + cs - 1
        ok = ok & (idx < QCAP)
        plsc.store_scatter(bqs, [idx], sv, mask=ok)
        plsc.store_scatter(bqd, [idx], loc, mask=ok)
        pq = pq + lax.reduce_max(cs, (0,))
      out_pos.append(jnp.minimum(pq, QCAP))
    return tuple(out_pos)

  z = jnp.int32(0)
  pos = lax.fori_loop(0, n_chunks, chunk, (z, z, z, z))
  for q in range(4):
    bqs, bqd = bufs[q]
    for k in range(2 * CH // 16):
      io = pos[q] + k * 16 + _iota16()
      plsc.store_scatter(bqs, [io], (k % 16) * 16 + _iota16())
      plsc.store_scatter(bqd, [io], Q + _iota16())
    nch = ((pos[q] + 2 * CH - 1) // (2 * CH)) * 2
    cv[...] = jnp.broadcast_to(nch, (16,)).astype(jnp.int32)
    pltpu.sync_copy(cv, cnts.at[q, t])
    o = pl.multiple_of((q * 32 + t) * EPTQ, 32)
    pltpu.sync_copy(bqs, psq.at[pl.ds(o, EPTQ)])
    pltpu.sync_copy(bqd, pdq.at[pl.ds(o, EPTQ)])


_partition = pl.kernel(
    _part_body,
    out_type=[jax.ShapeDtypeStruct((4 * 32 * EPTQ,), jnp.int32)] * 2
    + [jax.ShapeDtypeStruct((4, 32, 16), jnp.int32)],
    mesh=_MESH,
    compiler_params=_SC_PARAMS,
    scratch_types=[
        pltpu.VMEM((E_CH,), jnp.int32),
        pltpu.VMEM((E_CH,), jnp.int32),
    ] + [pltpu.VMEM((EPTQ,), jnp.int32)] * 8
    + [pltpu.VMEM((16,), jnp.int32)],
)


def _agg_layer_body(tgp, ps3, pd3, cnts, zq, tpg, pgs3, pgd3, zg,
                    outp, outg, sblk, dblk, rows0, rows1,
                    sem0, sem1, ssem0, ssem1, cv, acc):
  accq_rows = _acc_rows(Q)
  accg_rows = _acc_rows(N_G)
  c = lax.axis_index("c")
  s = lax.axis_index("s")
  t = c * NS + s
  rows = (rows0, rows1)
  sem = (sem0, sem1)
  ssem = (ssem0, ssem1)

  def region(q, rt):
    pltpu.sync_copy(cnts.at[q, rt], cv)
    nc = lax.reduce_max(cv[...], (0,))
    r0 = (q * 32 + rt) * QROWS
    pltpu.sync_copy((ps3.at[pl.ds(r0, QROWS)], pd3.at[pl.ds(r0, QROWS)]),
                    (sblk, dblk))

    @pl.when(nc > 0)
    def _():
      pltpu.async_copy(tgp.at[sblk.at[0]], rows[0], sem[0])

    def pair(i2, carry):
      i = i2 * 2
      for b in (0, 1):
        jj = i + b
        @pl.when(jj >= 1)
        def _():
          pltpu.make_async_copy(
              rows[1 - b], acc.at[dblk.at[jj - 1]], ssem[1 - b]).wait()
        @pl.when(jj + 1 < nc)
        def _():
          pltpu.async_copy(tgp.at[sblk.at[jj + 1]], rows[1 - b],
                           sem[1 - b])
        pltpu.make_async_copy(tgp.at[sblk.at[jj]], rows[b], sem[b]).wait()
        pltpu.make_async_copy(rows[b], acc.at[dblk.at[jj]],
                              ssem[b]).start(add=True)
      return carry

    lax.fori_loop(0, nc // 2, pair, 0)

    @pl.when(nc > 0)
    def _():
      pltpu.make_async_copy(
          rows[1], acc.at[dblk.at[nc - 1]], ssem[1]).wait()

  def p_phase(q):
    zrows = accq_rows // 16
    orows = Q // 16
    z0 = pl.multiple_of(s * zrows, zrows)
    pltpu.sync_copy(zq.at[pl.ds(z0, zrows)], acc.at[pl.ds(z0, zrows)])
    plsc.subcore_barrier()
    region(q, s)
    region(q, s + NS)
    plsc.subcore_barrier()
    o0 = pl.multiple_of(s * orows, orows)
    pltpu.sync_copy(acc.at[pl.ds(o0, orows)], outp.at[q, pl.ds(o0, orows)])
    plsc.subcore_barrier()

  @pl.when(c == 0)
  def _():
    p_phase(0)
    p_phase(1)

  @pl.when(c == 1)
  def _():
    p_phase(2)
    p_phase(3)

  zgr = accg_rows // 16
  zg0 = pl.multiple_of(s * zgr, zgr)
  pltpu.sync_copy(zg.at[pl.ds(zg0, zgr)], acc.at[pl.ds(zg0, zgr)])
  plsc.subcore_barrier()

  def seg(h0, hn):
    pltpu.sync_copy((pgs3.at[t, pl.ds(h0, hn)], pgd3.at[t, pl.ds(h0, hn)]),
                    (sblk.at[pl.ds(0, hn)], dblk.at[pl.ds(0, hn)]))
    pltpu.async_copy(tpg.at[sblk.at[0]], rows[0], sem[0])
    for j in range(hn):
      b = j % 2
      if j >= 1:
        pltpu.make_async_copy(
            rows[1 - b], acc.at[dblk.at[j - 1]], ssem[1 - b]).wait()
      if j + 1 < hn:
        pltpu.async_copy(tpg.at[sblk.at[j + 1]], rows[1 - b], sem[1 - b])
      pltpu.make_async_copy(tpg.at[sblk.at[j]], rows[b], sem[b]).wait()
      pltpu.make_async_copy(rows[b], acc.at[dblk.at[j]],
                            ssem[b]).start(add=True)

  for h0, hn in ((0, 24), (24, 24), (48, 8)):
    seg(h0, hn)
    pltpu.make_async_copy(rows[1], acc.at[dblk.at[hn - 1]], ssem[1]).wait()
  plsc.subcore_barrier()
  ogr = _out_rows(N_G) // 16
  og0 = pl.multiple_of(s * ogr, ogr)
  pltpu.sync_copy(acc.at[pl.ds(og0, ogr)], outg.at[c, pl.ds(og0, ogr)])


_agg_layer = pl.kernel(
    _agg_layer_body,
    out_type=[jax.ShapeDtypeStruct((4, Q, D), jnp.float32),
              jax.ShapeDtypeStruct((NC, _out_rows(N_G), D), jnp.float32)],
    mesh=_MESH,
    compiler_params=_SC_PARAMS,
    scratch_types=[
        pltpu.VMEM((QROWS, CH), jnp.int32),
        pltpu.VMEM((QROWS, CH), jnp.int32),
        pltpu.VMEM((CH, D), jnp.float32),
        pltpu.VMEM((CH, D), jnp.float32),
        pltpu.SemaphoreType.DMA,
        pltpu.SemaphoreType.DMA,
        pltpu.SemaphoreType.DMA,
        pltpu.SemaphoreType.DMA,
        pltpu.VMEM((16,), jnp.int32),
        pltpu.VMEM_SHARED((_acc_rows(Q), D), jnp.float32),
    ],
)


def _cnts_body(pd3, pgd3, cnts, zq, zg, ones, out_p, out_g,
               dblk, dall, ones_v, cv, sem, accp, accg):
  accq_rows = _acc_rows(Q)
  accg_rows = _acc_rows(N_G)
  c = lax.axis_index("c")
  s = lax.axis_index("s")
  t = c * NS + s
  zgr = accg_rows // 16
  zg0 = pl.multiple_of(s * zgr, zgr)
  pltpu.sync_copy(zg.at[pl.ds(zg0, zgr)], accg.at[pl.ds(zg0, zgr)])
  pltpu.sync_copy(ones, ones_v)
  plsc.subcore_barrier()

  ones_c = ones_v.at[pl.ds(0, CH)]
  pltpu.sync_copy(pgd3.at[t], dall)
  for r in range(0, G_CPT, 14):
    for i in range(r, r + 14):
      pltpu.make_async_copy(ones_c, accg.at[dall.at[i]], sem).start(add=True)
    for i in range(r, r + 14):
      pltpu.make_async_copy(ones_c, accg.at[dall.at[i]], sem).wait()

  zqr = accq_rows // 16
  opr = Q // 16

  def region(q, t):
    pltpu.sync_copy(cnts.at[q, t], cv)
    nc = lax.reduce_max(cv[...], (0,))
    r0 = (q * 32 + t) * QROWS
    pltpu.sync_copy(pd3.at[pl.ds(r0, QROWS)], dblk)

    def fire(i, carry):
      pltpu.make_async_copy(ones_c, accp.at[dblk.at[i]], sem).start(add=True)
      return carry

    def drain(i, carry):
      pltpu.make_async_copy(ones_c, accp.at[dblk.at[i]], sem).wait()
      return carry

    lax.fori_loop(0, nc, fire, 0)
    lax.fori_loop(0, nc, drain, 0)

  def phase(q):
    z0 = pl.multiple_of(s * zqr, zqr)
    pltpu.sync_copy(zq.at[pl.ds(z0, zqr)], accp.at[pl.ds(z0, zqr)])
    plsc.subcore_barrier()
    region(q, s)
    region(q, s + NS)
    plsc.subcore_barrier()
    o0 = pl.multiple_of(s * opr, opr)
    pltpu.sync_copy(accp.at[pl.ds(o0, opr)], out_p.at[q, pl.ds(o0, opr)])
    plsc.subcore_barrier()

  @pl.when(c == 0)
  def _():
    phase(0)
    phase(1)

  @pl.when(c == 1)
  def _():
    phase(2)
    phase(3)

  plsc.subcore_barrier()
  ogr = _out_rows(N_G) // 16
  og0 = pl.multiple_of(s * ogr, ogr)
  pltpu.sync_copy(accg.at[pl.ds(og0, ogr)], out_g.at[c, pl.ds(og0, ogr)])


_counts_all = pl.kernel(
    _cnts_body,
    out_type=[jax.ShapeDtypeStruct((4, Q, 8), jnp.float32),
              jax.ShapeDtypeStruct((NC, _out_rows(N_G), 8), jnp.float32)],
    mesh=_MESH,
    compiler_params=_SC_PARAMS,
    scratch_types=[
        pltpu.VMEM((QROWS, CH), jnp.int32),
        pltpu.VMEM((G_CPT, CH), jnp.int32),
        pltpu.VMEM((E_CH, 8), jnp.float32),
        pltpu.VMEM((16,), jnp.int32),
        pltpu.SemaphoreType.DMA,
        pltpu.VMEM_SHARED((_acc_rows(Q), 8), jnp.float32),
        pltpu.VMEM_SHARED((_acc_rows(N_G), 8), jnp.float32),
    ],
)


def _cls_body(xp, xg, ls, ld, out,
              sidx0, sidx1, didx0, didx1, rp0, rp1, rg0, rg1,
              semp0, semp1, semg0, semg1, tb, ov):
  c = lax.axis_index("c")
  s = lax.axis_index("s")
  base = (c * NS + s) * L_TILE
  ridx = _iota16() * 16
  n_chunks = L_TILE // L_CH
  sidx = (sidx0, sidx1)
  didx = (didx0, didx1)
  rp = (rp0, rp1)
  rg = (rg0, rg1)
  semp = (semp0, semp1)
  semg = (semg0, semg1)

  def load(k, b):
    cb = pl.multiple_of(base + k * L_CH, 32)
    pltpu.sync_copy((ls.at[pl.ds(cb, L_CH)], ld.at[pl.ds(cb, L_CH)]),
                    (sidx[b], didx[b]))
    pltpu.async_copy(xp.at[sidx[b]], rp[b], semp[b])
    pltpu.async_copy(xg.at[didx[b]], rg[b], semg[b])

  load(0, 0)
  for k in range(n_chunks):
    b = k % 2
    if k + 1 < n_chunks:
      load(k + 1, 1 - b)
    pltpu.make_async_copy(xp.at[sidx[b]], rp[b], semp[b]).wait()
    pltpu.make_async_copy(xg.at[didx[b]], rg[b], semg[b]).wait()
    cb = pl.multiple_of(base + k * L_CH, 32)

    def g16(g, carry):
      for j in range(16):
        r = g * 16 + j
        acc = None
        for m in range(4):
          a = rp[b][r, pl.ds(16 * m, 16)]
          v = rg[b][r, pl.ds(16 * m, 16)]
          av = a * v
          acc = av if acc is None else acc + av
        tb[pl.ds(j * 16, 16)] = acc
      tot = jnp.zeros((16,), jnp.float32)
      for m in range(16):
        tot = tot + plsc.load_gather(tb, [ridx + m])
      ov[pl.ds(g * 16, 16)] = tot
      return carry

    lax.fori_loop(0, L_CH // 16, g16, 0)
    pltpu.sync_copy(ov, out.at[pl.ds(cb, L_CH)])


_classifier = pl.kernel(
    _cls_body,
    out_type=jax.ShapeDtypeStruct((L_PAD,), jnp.float32),
    mesh=_MESH,
    compiler_params=_SC_PARAMS,
    scratch_types=[
        pltpu.VMEM((L_CH,), jnp.int32),
        pltpu.VMEM((L_CH,), jnp.int32),
        pltpu.VMEM((L_CH,), jnp.int32),
        pltpu.VMEM((L_CH,), jnp.int32),
        pltpu.VMEM((L_CH, D), jnp.float32),
        pltpu.VMEM((L_CH, D), jnp.float32),
        pltpu.VMEM((L_CH, D), jnp.float32),
        pltpu.VMEM((L_CH, D), jnp.float32),
        pltpu.SemaphoreType.DMA,
        pltpu.SemaphoreType.DMA,
        pltpu.SemaphoreType.DMA,
        pltpu.SemaphoreType.DMA,
        pltpu.VMEM((256,), jnp.float32),
        pltpu.VMEM((L_CH,), jnp.float32),
    ],
)


def _init_xg_body(gx, w, b, ge, out):
  acc = lax.dot_general(gx[...], w[...], (((1,), (1,)), ((), ())),
                        preferred_element_type=jnp.float32)
  out[...] = acc + b[...] + ge[...]


def _init_xg(gx, w, b2, ge):
  blk = 1000
  return pl.pallas_call(
      _init_xg_body,
      grid=(N_G // blk,),
      in_specs=[
          pl.BlockSpec((blk, 1000), lambda i: (i, 0)),
          pl.BlockSpec((D, 1000), lambda i: (0, 0)),
          pl.BlockSpec((1, D), lambda i: (0, 0)),
          pl.BlockSpec((blk, D), lambda i: (i, 0)),
      ],
      out_specs=pl.BlockSpec((blk, D), lambda i: (i, 0)),
      out_shape=jax.ShapeDtypeStruct((N_G, D), jnp.float32),
  )(gx, w, b2, ge)


def _make_transform_body(planes, relu):
  def body(x, agg, cnt, wl, wr, b, out):
    a = agg[0]
    n = cnt[0, :, 0:1]
    for p in range(1, planes):
      a = a + agg[p]
      n = n + cnt[p, :, 0:1]
    mean = a / jnp.maximum(n, 1.0)
    o = (lax.dot_general(mean, wl[...], (((1,), (1,)), ((), ())),
                         preferred_element_type=jnp.float32)
         + lax.dot_general(x[...], wr[...], (((1,), (1,)), ((), ())),
                           preferred_element_type=jnp.float32)
         + b[...])
    if relu:
      o = jnp.maximum(o, 0.0)
    out[...] = o
  return body


def _transform(x, agg, cnt, wl, wr, b2, relu, blk):
  planes = agg.shape[0]
  rows = x.shape[0]
  return pl.pallas_call(
      _make_transform_body(planes, relu),
      grid=(rows // blk,),
      in_specs=[
          pl.BlockSpec((blk, D), lambda i: (i, 0)),
          pl.BlockSpec((planes, blk, D), lambda i: (0, i, 0)),
          pl.BlockSpec((planes, blk, 8), lambda i: (0, i, 0)),
          pl.BlockSpec((D, D), lambda i: (0, 0)),
          pl.BlockSpec((D, D), lambda i: (0, 0)),
          pl.BlockSpec((1, D), lambda i: (0, 0)),
      ],
      out_specs=pl.BlockSpec((blk, D), lambda i: (i, 0)),
      out_shape=jax.ShapeDtypeStruct((rows, D), jnp.float32),
  )(x, agg, cnt, wl, wr, b2)


def kernel(protein_n_id, go_term_n_id, go_term_x, e_gp_src, e_gp_dst,
           e_pg_src, e_pg_dst, label_src, label_dst, protein_emb,
           go_term_emb, lin_W, lin_b, Wl, bl, Wr):
  f32 = jnp.float32
  xp = jnp.concatenate(
      [protein_emb, jnp.zeros((P_PAD - N_P, D), f32)], axis=0)
  xg = _init_xg(go_term_x, lin_W, lin_b.reshape(1, D), go_term_emb)

  epad = E_PAD - E
  zpad_i = jnp.zeros((epad,), jnp.int32)
  gp_s = jnp.concatenate([e_gp_src, zpad_i])
  gp_d = jnp.concatenate([e_gp_dst, jnp.full((epad,), -1, jnp.int32)])
  tpad_i = N_G + (jnp.arange(epad, dtype=jnp.int32) % 16)
  pg_s = jnp.concatenate([e_pg_src, zpad_i])
  pg_d = jnp.concatenate([e_pg_dst, tpad_i])
  pgs3 = pg_s.reshape(NC * NS, G_CPT, CH)
  pgd3 = pg_d.reshape(NC * NS, G_CPT, CH)

  z64_q = jnp.zeros((_acc_rows(Q), D), f32)
  z64_g = jnp.zeros((_acc_rows(N_G), D), f32)
  z8_q = jnp.zeros((_acc_rows(Q), 8), f32)
  z8_g = jnp.zeros((_acc_rows(N_G), 8), f32)
  ones8 = jnp.ones((E_CH, 8), f32)

  psq, pdq, pcnts = _partition(gp_s, gp_d)
  ps3 = psq.reshape(4 * 32 * QROWS, CH)
  pd3 = pdq.reshape(4 * 32 * QROWS, CH)
  cnt_p, cnt_g = _counts_all(pd3, pgd3, pcnts, z8_q, z8_g, ones8)
  cnt_p = cnt_p.reshape(1, P_PAD, 8)

  for layer in range(3):
    relu = layer < 2
    agg_p, agg_g = _agg_layer(xg, ps3, pd3, pcnts, z64_q,
                              xp, pgs3, pgd3, z64_g)
    agg_p = agg_p.reshape(1, P_PAD, D)
    xp = _transform(xp, agg_p, cnt_p, Wl[2 * layer], Wr[2 * layer],
                    bl[2 * layer].reshape(1, D), relu, 512)
    xg = _transform(xg, agg_g, cnt_g, Wl[2 * layer + 1], Wr[2 * layer + 1],
                    bl[2 * layer + 1].reshape(1, D), relu, 1000)

  lpad = L_PAD - L
  ls = jnp.concatenate([label_src, jnp.zeros((lpad,), jnp.int32)])
  ld = jnp.concatenate([label_dst, jnp.zeros((lpad,), jnp.int32)])
  pred = _classifier(xp, xg, ls, ld)
  return pred[:L]


# speedup vs baseline: 1.2474x; 1.2474x over previous
import jax
import jax.numpy as jnp
from jax import lax
from jax.experimental import pallas as pl
from jax.experimental.pallas import tpu as pltpu
from jax.experimental.pallas import tpu_sc as plsc

N_P, N_G, D, E, L = 50000, 10000, 64, 800000, 100000

NC, NS = 2, 16
P_PAD = 50176
Q = P_PAD // 4
E_CH = 512
E_PAD = 802816
CH = 448
G_CPT = E_PAD // (NC * NS * CH)
QROWS = 24
EPTQ = QROWS * CH
QCAP = EPTQ - 2 * CH
L_TILE = 3136
L_CH = 448
L_PAD = 32 * L_TILE

_MESH = plsc.VectorSubcoreMesh(
    core_axis_name="c", subcore_axis_name="s", num_cores=NC, num_subcores=NS)
_SC_PARAMS = pltpu.CompilerParams(
    use_tc_tiling_on_sc=False, needs_layout_passes=False)


def _iota16():
  return lax.iota(jnp.int32, 16)


def _acc_rows(own):
  return -(-(own + 16) // 128) * 128


def _out_rows(own):
  r = own // 16
  return own if r % 8 == 0 else _acc_rows(own)


def _make_agg_g():
  own = N_G
  acc_rows = _acc_rows(own)
  out_rows = _out_rows(own)
  zrows = acc_rows // 16
  orows = out_rows // 16

  def body(table, src3, dst3, zinit, out,
           sblk, dblk, rows0, rows1, sem0, sem1, ssem0, ssem1, acc):
    c = lax.axis_index("c")
    s = lax.axis_index("s")
    z0 = pl.multiple_of(s * zrows, zrows)
    pltpu.sync_copy(zinit.at[pl.ds(z0, zrows)], acc.at[pl.ds(z0, zrows)])
    plsc.subcore_barrier()
    t = c * NS + s
    rows = (rows0, rows1)
    sem = (sem0, sem1)
    ssem = (ssem0, ssem1)

    def seg(h0, hn):
      pltpu.sync_copy((src3.at[t, pl.ds(h0, hn)], dst3.at[t, pl.ds(h0, hn)]),
                      (sblk.at[pl.ds(0, hn)], dblk.at[pl.ds(0, hn)]))
      pltpu.async_copy(table.at[sblk.at[0]], rows[0], sem[0])
      for j in range(hn):
        b = j % 2
        if j >= 1:
          pltpu.make_async_copy(
              rows[1 - b], acc.at[dblk.at[j - 1]], ssem[1 - b]).wait()
        if j + 1 < hn:
          pltpu.async_copy(table.at[sblk.at[j + 1]], rows[1 - b], sem[1 - b])
        pltpu.make_async_copy(table.at[sblk.at[j]], rows[b], sem[b]).wait()
        pltpu.make_async_copy(rows[b], acc.at[dblk.at[j]],
                              ssem[b]).start(add=True)

    seg(0, 32)
    pltpu.make_async_copy(rows[1], acc.at[dblk.at[31]], ssem[1]).wait()
    seg(32, 24)
    pltpu.make_async_copy(rows[1], acc.at[dblk.at[23]], ssem[1]).wait()
    plsc.subcore_barrier()
    o0 = pl.multiple_of(s * orows, orows)
    pltpu.sync_copy(acc.at[pl.ds(o0, orows)], out.at[c, pl.ds(o0, orows)])

  return pl.kernel(
      body,
      out_type=jax.ShapeDtypeStruct((NC, out_rows, D), jnp.float32),
      mesh=_MESH,
      compiler_params=_SC_PARAMS,
      scratch_types=[
          pltpu.VMEM((32, CH), jnp.int32),
          pltpu.VMEM((32, CH), jnp.int32),
          pltpu.VMEM((CH, D), jnp.float32),
          pltpu.VMEM((CH, D), jnp.float32),
          pltpu.SemaphoreType.DMA,
          pltpu.SemaphoreType.DMA,
          pltpu.SemaphoreType.DMA,
          pltpu.SemaphoreType.DMA,
          pltpu.VMEM_SHARED((acc_rows, D), jnp.float32),
      ],
  )


_agg_g = _make_agg_g()


def _part_body(srcp, dstp, psq, pdq, cnts,
               sidx, didx, bs0, bd0, bs1, bd1, bs2, bd2, bs3, bd3, cv):
  c = lax.axis_index("c")
  s = lax.axis_index("s")
  t = c * NS + s
  ept = E_PAD // 32
  ebase = t * ept
  n_chunks = ept // E_CH
  bufs = ((bs0, bd0), (bs1, bd1), (bs2, bd2), (bs3, bd3))

  def chunk(i, pos):
    eb = pl.multiple_of(ebase + i * E_CH, 32)
    pltpu.sync_copy((srcp.at[pl.ds(eb, E_CH)], dstp.at[pl.ds(eb, E_CH)]),
                    (sidx, didx))
    out_pos = []
    for q in range(4):
      pq = pos[q]
      bqs, bqd = bufs[q]
      for j in range(E_CH // 16):
        sv = sidx[pl.ds(j * 16, 16)]
        dv = didx[pl.ds(j * 16, 16)]
        loc = dv - q * Q
        ok = (loc >= 0) & (loc < Q)
        cs = plsc.cumsum(jnp.where(ok, 1, 0))
        idx = pq + cs - 1
        ok = ok & (idx < QCAP)
        plsc.store_scatter(bqs, [idx], sv, mask=ok)
        plsc.store_scatter(bqd, [idx], loc, mask=ok)
        pq = pq + plsc.all_reduce_population_count(ok)
      out_pos.append(jnp.minimum(pq, QCAP))
    return tuple(out_pos)

  zv = jnp.zeros((16,), jnp.int32)
  pos = lax.fori_loop(0, n_chunks, chunk, (zv, zv, zv, zv))
  pos = tuple(lax.reduce_max(p, (0,)) for p in pos)
  for q in range(4):
    bqs, bqd = bufs[q]
    for k in range(2 * CH // 16):
      io = pos[q] + k * 16 + _iota16()
      plsc.store_scatter(bqs, [io], (k % 16) * 16 + _iota16())
      plsc.store_scatter(bqd, [io], Q + _iota16())
    nch = ((pos[q] + 2 * CH - 1) // (2 * CH)) * 2
    cv[...] = jnp.broadcast_to(nch, (16,)).astype(jnp.int32)
    pltpu.sync_copy(cv, cnts.at[q, t])
    o = pl.multiple_of((q * 32 + t) * EPTQ, 32)
    pltpu.sync_copy(bqs, psq.at[pl.ds(o, EPTQ)])
    pltpu.sync_copy(bqd, pdq.at[pl.ds(o, EPTQ)])


_partition = pl.kernel(
    _part_body,
    out_type=[jax.ShapeDtypeStruct((4 * 32 * EPTQ,), jnp.int32)] * 2
    + [jax.ShapeDtypeStruct((4, 32, 16), jnp.int32)],
    mesh=_MESH,
    compiler_params=_SC_PARAMS,
    scratch_types=[
        pltpu.VMEM((E_CH,), jnp.int32),
        pltpu.VMEM((E_CH,), jnp.int32),
    ] + [pltpu.VMEM((EPTQ,), jnp.int32)] * 8
    + [pltpu.VMEM((16,), jnp.int32)],
)


def _agg_pq_body(table, ps3, pd3, cnts, zinit, out,
                 sblk, dblk, rows0, rows1, sem0, sem1, ssem0, ssem1,
                 cv, acc):
  acc_rows = _acc_rows(Q)
  zrows = acc_rows // 16
  orows = Q // 16
  c = lax.axis_index("c")
  s = lax.axis_index("s")
  rows = (rows0, rows1)
  sem = (sem0, sem1)
  ssem = (ssem0, ssem1)

  def region(q, t):
    pltpu.sync_copy(cnts.at[q, t], cv)
    nc = lax.reduce_max(cv[...], (0,))
    r0 = (q * 32 + t) * QROWS
    pltpu.sync_copy((ps3.at[pl.ds(r0, QROWS)], pd3.at[pl.ds(r0, QROWS)]),
                    (sblk, dblk))

    @pl.when(nc > 0)
    def _():
      pltpu.async_copy(table.at[sblk.at[0]], rows[0], sem[0])

    def pair(i2, carry):
      i = i2 * 2
      for b in (0, 1):
        jj = i + b
        @pl.when(jj >= 1)
        def _():
          pltpu.make_async_copy(
              rows[1 - b], acc.at[dblk.at[jj - 1]], ssem[1 - b]).wait()
        @pl.when(jj + 1 < nc)
        def _():
          pltpu.async_copy(table.at[sblk.at[jj + 1]], rows[1 - b],
                           sem[1 - b])
        pltpu.make_async_copy(table.at[sblk.at[jj]], rows[b], sem[b]).wait()
        pltpu.make_async_copy(rows[b], acc.at[dblk.at[jj]],
                              ssem[b]).start(add=True)
      return carry

    lax.fori_loop(0, nc // 2, pair, 0)

    @pl.when(nc > 0)
    def _():
      pltpu.make_async_copy(
          rows[1], acc.at[dblk.at[nc - 1]], ssem[1]).wait()

  def phase(q):
    z0 = pl.multiple_of(s * zrows, zrows)
    pltpu.sync_copy(zinit.at[pl.ds(z0, zrows)], acc.at[pl.ds(z0, zrows)])
    plsc.subcore_barrier()
    region(q, s)
    region(q, s + NS)
    plsc.subcore_barrier()
    o0 = pl.multiple_of(s * orows, orows)
    pltpu.sync_copy(acc.at[pl.ds(o0, orows)], out.at[q, pl.ds(o0, orows)])
    plsc.subcore_barrier()

  @pl.when(c == 0)
  def _():
    phase(0)
    phase(1)

  @pl.when(c == 1)
  def _():
    phase(2)
    phase(3)


_agg_pq = pl.kernel(
    _agg_pq_body,
    out_type=jax.ShapeDtypeStruct((4, Q, D), jnp.float32),
    mesh=_MESH,
    compiler_params=_SC_PARAMS,
    scratch_types=[
        pltpu.VMEM((QROWS, CH), jnp.int32),
        pltpu.VMEM((QROWS, CH), jnp.int32),
        pltpu.VMEM((CH, D), jnp.float32),
        pltpu.VMEM((CH, D), jnp.float32),
        pltpu.SemaphoreType.DMA,
        pltpu.SemaphoreType.DMA,
        pltpu.SemaphoreType.DMA,
        pltpu.SemaphoreType.DMA,
        pltpu.VMEM((16,), jnp.int32),
        pltpu.VMEM_SHARED((_acc_rows(Q), D), jnp.float32),
    ],
)


def _cnts_body(pd3, pgd3, cnts, zq, zg, ones, out_p, out_g,
               dblk, dall, ones_v, cv, sem, accp, accg):
  accq_rows = _acc_rows(Q)
  accg_rows = _acc_rows(N_G)
  c = lax.axis_index("c")
  s = lax.axis_index("s")
  t = c * NS + s
  zgr = accg_rows // 16
  zg0 = pl.multiple_of(s * zgr, zgr)
  pltpu.sync_copy(zg.at[pl.ds(zg0, zgr)], accg.at[pl.ds(zg0, zgr)])
  pltpu.sync_copy(ones, ones_v)
  plsc.subcore_barrier()

  ones_c = ones_v.at[pl.ds(0, CH)]
  pltpu.sync_copy(pgd3.at[t], dall)
  for r in range(0, G_CPT, 14):
    for i in range(r, r + 14):
      pltpu.make_async_copy(ones_c, accg.at[dall.at[i]], sem).start(add=True)
    for i in range(r, r + 14):
      pltpu.make_async_copy(ones_c, accg.at[dall.at[i]], sem).wait()

  zqr = accq_rows // 16
  opr = Q // 16

  def region(q, t):
    pltpu.sync_copy(cnts.at[q, t], cv)
    nc = lax.reduce_max(cv[...], (0,))
    r0 = (q * 32 + t) * QROWS
    pltpu.sync_copy(pd3.at[pl.ds(r0, QROWS)], dblk)

    def fire(i, carry):
      pltpu.make_async_copy(ones_c, accp.at[dblk.at[i]], sem).start(add=True)
      return carry

    def drain(i, carry):
      pltpu.make_async_copy(ones_c, accp.at[dblk.at[i]], sem).wait()
      return carry

    lax.fori_loop(0, nc, fire, 0)
    lax.fori_loop(0, nc, drain, 0)

  def phase(q):
    z0 = pl.multiple_of(s * zqr, zqr)
    pltpu.sync_copy(zq.at[pl.ds(z0, zqr)], accp.at[pl.ds(z0, zqr)])
    plsc.subcore_barrier()
    region(q, s)
    region(q, s + NS)
    plsc.subcore_barrier()
    o0 = pl.multiple_of(s * opr, opr)
    pltpu.sync_copy(accp.at[pl.ds(o0, opr)], out_p.at[q, pl.ds(o0, opr)])
    plsc.subcore_barrier()

  @pl.when(c == 0)
  def _():
    phase(0)
    phase(1)

  @pl.when(c == 1)
  def _():
    phase(2)
    phase(3)

  plsc.subcore_barrier()
  ogr = _out_rows(N_G) // 16
  og0 = pl.multiple_of(s * ogr, ogr)
  pltpu.sync_copy(accg.at[pl.ds(og0, ogr)], out_g.at[c, pl.ds(og0, ogr)])


_counts_all = pl.kernel(
    _cnts_body,
    out_type=[jax.ShapeDtypeStruct((4, Q, 8), jnp.float32),
              jax.ShapeDtypeStruct((NC, _out_rows(N_G), 8), jnp.float32)],
    mesh=_MESH,
    compiler_params=_SC_PARAMS,
    scratch_types=[
        pltpu.VMEM((QROWS, CH), jnp.int32),
        pltpu.VMEM((G_CPT, CH), jnp.int32),
        pltpu.VMEM((E_CH, 8), jnp.float32),
        pltpu.VMEM((16,), jnp.int32),
        pltpu.SemaphoreType.DMA,
        pltpu.VMEM_SHARED((_acc_rows(Q), 8), jnp.float32),
        pltpu.VMEM_SHARED((_acc_rows(N_G), 8), jnp.float32),
    ],
)


def _cls_body(xp, xg, ls, ld, out,
              sidx0, sidx1, didx0, didx1, rp0, rp1, rg0, rg1,
              semp0, semp1, semg0, semg1, tb, ov):
  c = lax.axis_index("c")
  s = lax.axis_index("s")
  base = (c * NS + s) * L_TILE
  ridx = _iota16() * 16
  n_chunks = L_TILE // L_CH
  sidx = (sidx0, sidx1)
  didx = (didx0, didx1)
  rp = (rp0, rp1)
  rg = (rg0, rg1)
  semp = (semp0, semp1)
  semg = (semg0, semg1)

  def load(k, b):
    cb = pl.multiple_of(base + k * L_CH, 32)
    pltpu.sync_copy((ls.at[pl.ds(cb, L_CH)], ld.at[pl.ds(cb, L_CH)]),
                    (sidx[b], didx[b]))
    pltpu.async_copy(xp.at[sidx[b]], rp[b], semp[b])
    pltpu.async_copy(xg.at[didx[b]], rg[b], semg[b])

  load(0, 0)
  for k in range(n_chunks):
    b = k % 2
    if k + 1 < n_chunks:
      load(k + 1, 1 - b)
    pltpu.make_async_copy(xp.at[sidx[b]], rp[b], semp[b]).wait()
    pltpu.make_async_copy(xg.at[didx[b]], rg[b], semg[b]).wait()
    cb = pl.multiple_of(base + k * L_CH, 32)

    def g16(g, carry):
      for j in range(16):
        r = g * 16 + j
        acc = None
        for m in range(4):
          a = rp[b][r, pl.ds(16 * m, 16)]
          v = rg[b][r, pl.ds(16 * m, 16)]
          av = a * v
          acc = av if acc is None else acc + av
        tb[pl.ds(j * 16, 16)] = acc
      tot = jnp.zeros((16,), jnp.float32)
      for m in range(16):
        tot = tot + plsc.load_gather(tb, [ridx + m])
      ov[pl.ds(g * 16, 16)] = tot
      return carry

    lax.fori_loop(0, L_CH // 16, g16, 0)
    pltpu.sync_copy(ov, out.at[pl.ds(cb, L_CH)])


_classifier = pl.kernel(
    _cls_body,
    out_type=jax.ShapeDtypeStruct((L_PAD,), jnp.float32),
    mesh=_MESH,
    compiler_params=_SC_PARAMS,
    scratch_types=[
        pltpu.VMEM((L_CH,), jnp.int32),
        pltpu.VMEM((L_CH,), jnp.int32),
        pltpu.VMEM((L_CH,), jnp.int32),
        pltpu.VMEM((L_CH,), jnp.int32),
        pltpu.VMEM((L_CH, D), jnp.float32),
        pltpu.VMEM((L_CH, D), jnp.float32),
        pltpu.VMEM((L_CH, D), jnp.float32),
        pltpu.VMEM((L_CH, D), jnp.float32),
        pltpu.SemaphoreType.DMA,
        pltpu.SemaphoreType.DMA,
        pltpu.SemaphoreType.DMA,
        pltpu.SemaphoreType.DMA,
        pltpu.VMEM((256,), jnp.float32),
        pltpu.VMEM((L_CH,), jnp.float32),
    ],
)


def _init_xg_body(gx, w, b, ge, out):
  acc = lax.dot_general(gx[...], w[...], (((1,), (1,)), ((), ())),
                        preferred_element_type=jnp.float32)
  out[...] = acc + b[...] + ge[...]


def _init_xg(gx, w, b2, ge):
  blk = 1000
  return pl.pallas_call(
      _init_xg_body,
      grid=(N_G // blk,),
      in_specs=[
          pl.BlockSpec((blk, 1000), lambda i: (i, 0)),
          pl.BlockSpec((D, 1000), lambda i: (0, 0)),
          pl.BlockSpec((1, D), lambda i: (0, 0)),
          pl.BlockSpec((blk, D), lambda i: (i, 0)),
      ],
      out_specs=pl.BlockSpec((blk, D), lambda i: (i, 0)),
      out_shape=jax.ShapeDtypeStruct((N_G, D), jnp.float32),
  )(gx, w, b2, ge)


def _make_transform_body(planes, relu):
  def body(x, agg, cnt, wl, wr, b, out):
    a = agg[0]
    n = cnt[0, :, 0:1]
    for p in range(1, planes):
      a = a + agg[p]
      n = n + cnt[p, :, 0:1]
    mean = a / jnp.maximum(n, 1.0)
    o = (lax.dot_general(mean, wl[...], (((1,), (1,)), ((), ())),
                         preferred_element_type=jnp.float32)
         + lax.dot_general(x[...], wr[...], (((1,), (1,)), ((), ())),
                           preferred_element_type=jnp.float32)
         + b[...])
    if relu:
      o = jnp.maximum(o, 0.0)
    out[...] = o
  return body


def _transform(x, agg, cnt, wl, wr, b2, relu, blk):
  planes = agg.shape[0]
  rows = x.shape[0]
  return pl.pallas_call(
      _make_transform_body(planes, relu),
      grid=(rows // blk,),
      in_specs=[
          pl.BlockSpec((blk, D), lambda i: (i, 0)),
          pl.BlockSpec((planes, blk, D), lambda i: (0, i, 0)),
          pl.BlockSpec((planes, blk, 8), lambda i: (0, i, 0)),
          pl.BlockSpec((D, D), lambda i: (0, 0)),
          pl.BlockSpec((D, D), lambda i: (0, 0)),
          pl.BlockSpec((1, D), lambda i: (0, 0)),
      ],
      out_specs=pl.BlockSpec((blk, D), lambda i: (i, 0)),
      out_shape=jax.ShapeDtypeStruct((rows, D), jnp.float32),
  )(x, agg, cnt, wl, wr, b2)


def kernel(protein_n_id, go_term_n_id, go_term_x, e_gp_src, e_gp_dst,
           e_pg_src, e_pg_dst, label_src, label_dst, protein_emb,
           go_term_emb, lin_W, lin_b, Wl, bl, Wr):
  f32 = jnp.float32
  xp = jnp.concatenate(
      [protein_emb, jnp.zeros((P_PAD - N_P, D), f32)], axis=0)
  xg = _init_xg(go_term_x, lin_W, lin_b.reshape(1, D), go_term_emb)

  epad = E_PAD - E
  zpad_i = jnp.zeros((epad,), jnp.int32)
  gp_s = jnp.concatenate([e_gp_src, zpad_i])
  gp_d = jnp.concatenate([e_gp_dst, jnp.full((epad,), -1, jnp.int32)])
  tpad_i = N_G + (jnp.arange(epad, dtype=jnp.int32) % 16)
  pg_s = jnp.concatenate([e_pg_src, zpad_i])
  pg_d = jnp.concatenate([e_pg_dst, tpad_i])
  pgs3 = pg_s.reshape(NC * NS, G_CPT, CH)
  pgd3 = pg_d.reshape(NC * NS, G_CPT, CH)

  z64_q = jnp.zeros((_acc_rows(Q), D), f32)
  z64_g = jnp.zeros((_acc_rows(N_G), D), f32)
  z8_q = jnp.zeros((_acc_rows(Q), 8), f32)
  z8_g = jnp.zeros((_acc_rows(N_G), 8), f32)
  ones8 = jnp.ones((E_CH, 8), f32)

  psq, pdq, pcnts = _partition(gp_s, gp_d)
  ps3 = psq.reshape(4 * 32 * QROWS, CH)
  pd3 = pdq.reshape(4 * 32 * QROWS, CH)
  cnt_p, cnt_g = _counts_all(pd3, pgd3, pcnts, z8_q, z8_g, ones8)
  cnt_p = cnt_p.reshape(1, P_PAD, 8)

  for layer in range(3):
    relu = layer < 2
    agg_p = _agg_pq(xg, ps3, pd3, pcnts, z64_q).reshape(1, P_PAD, D)
    agg_g = _agg_g(xp, pgs3, pgd3, z64_g)
    xp = _transform(xp, agg_p, cnt_p, Wl[2 * layer], Wr[2 * layer],
                    bl[2 * layer].reshape(1, D), relu, 512)
    xg = _transform(xg, agg_g, cnt_g, Wl[2 * layer + 1], Wr[2 * layer + 1],
                    bl[2 * layer + 1].reshape(1, D), relu, 1000)

  lpad = L_PAD - L
  ls = jnp.concatenate([label_src, jnp.zeros((lpad,), jnp.int32)])
  ld = jnp.concatenate([label_dst, jnp.zeros((lpad,), jnp.int32)])
  pred = _classifier(xp, xg, ls, ld)
  return pred[:L]
